# baseline (speedup 1.0000x reference)
import numpy as np
import jax
import jax.numpy as jnp
from jax.experimental import pallas as pl
from jax.experimental.pallas import tpu as pltpu


def _tap_weights(w_mat, S, Cin, Cout, ci_major):
    W = 2 * S + 2
    Kp = 4 * W * Cin
    P = np.zeros((4, Kp, S * 9 * Cin), np.float32)
    for t, (dh, dw) in enumerate([(0, 0), (0, 1), (1, 0), (1, 1)]):
        for s in range(S):
            for kh in range(3):
                for kw in range(3):
                    for ci in range(Cin):
                        r = dh + kh
                        c = 2 * s + dw + kw
                        if ci_major:
                            dst = (r * Cin + ci) * W + c
                        else:
                            dst = (r * W + c) * Cin + ci
                        src = (kh * 3 + kw) * Cin + ci
                        P[t, dst, s * 9 * Cin + src] = 1.0
    wrep = (jnp.eye(S, dtype=w_mat.dtype)[:, None, :, None]
            * w_mat[None, :, None, :]).reshape(S * 9 * Cin, S * Cout)
    wall = jnp.einsum('tkm,mn->tkn', jnp.asarray(P), wrep).astype(
        jnp.bfloat16 if Cin > 3 else jnp.float32)
    return [jnp.concatenate([wall[0], wall[1]], axis=1),
            jnp.concatenate([wall[2], wall[3]], axis=1)]


def _tap_dots_epilogue(pf_ref, wa_ref, wb_ref, b_ref, o_ref):
    p = pf_ref[...]
    za = jnp.dot(p, wa_ref[...], preferred_element_type=jnp.float32)
    zb = jnp.dot(p, wb_ref[...], preferred_element_type=jnp.float32)
    z = jnp.maximum(za, zb)
    z = jnp.maximum(z[:, :128], z[:, 128:])
    o_ref[...] = jnp.maximum(z + b_ref[...], 0.0).astype(o_ref.dtype)


_PHASE1 = {(ph, r): ((2 * ph + r - 1) % 4, (2 * ph + r - 1 - (2 * ph + r - 1) % 4) // 4)
           for ph in range(2) for r in range(4)}
_PHASE2 = {r: ((r - 1) % 2, (r - 1 - (r - 1) % 2) // 2) for r in range(4)}


def _conv1_patches(x_ref, pf_ref):
    IB = x_ref.shape[0]
    pf_ref[...] = jnp.zeros_like(pf_ref)
    for img in range(IB):
        planes = {}
        for ci in range(3):
            pb = x_ref[img, ci]
            pbm = jnp.concatenate(
                [pb, jnp.zeros((224, 32), pb.dtype)], axis=1)
            pbm = pbm.reshape(56, 1024)
            for pp in range(4):
                planes[(ci, pp)] = pbm[:, pp * 256:pp * 256 + 224]
        for g in range(14):
            cs = 16 * g - 1
            src_lo, src_hi = max(cs, 0), min(cs + 18, 224)
            dst_lo = src_lo - cs
            for ph in range(2):
                for r in range(4):
                    pp, dlt = _PHASE1[(ph, r)]
                    if dlt == 0:
                        qs, qe, ds, de = 0, 56, 0, 56
                    elif dlt < 0:
                        qs, qe, ds, de = 0, 55, 1, 56
                    else:
                        qs, qe, ds, de = 1, 56, 0, 55
                    rb = img * 1568 + g * 112 + ph * 56
                    for ci in range(3):
                        lane = (r * 3 + ci) * 18 + dst_lo
                        pf_ref[rb + ds:rb + de, lane:lane + src_hi - src_lo] = \
                            planes[(ci, pp)][qs:qe, src_lo:src_hi]


def _conv2_patches(y_ref, pf_ref):
    IB = y_ref.shape[0] // 1568
    pf_ref[...] = jnp.zeros_like(pf_ref)
    for img in range(IB):
        for g2 in range(14):
            for r in range(4):
                ph, dlt = _PHASE2[r]
                if dlt == 0:
                    qs, qe, ds, de = 0, 56, 0, 56
                elif dlt < 0:
                    qs, qe, ds, de = 0, 55, 1, 56
                else:
                    qs, qe, ds, de = 1, 56, 0, 55
                rb = img * 784 + g2 * 56
                lane0 = r * 160
                pieces = []
                if g2 > 0:
                    pieces.append((g2 - 1, 112, 16, lane0))
                pieces.append((g2, 0, 128, lane0 + 16))
                if g2 < 13:
                    pieces.append((g2 + 1, 0, 16, lane0 + 144))
                for (gs, ls, lw, dl) in pieces:
                    sb = img * 1568 + gs * 112 + ph * 56
                    pf_ref[rb + ds:rb + de, dl:dl + lw] = \
                        y_ref[sb + qs:sb + qe, ls:ls + lw]


def _convs_body(x_ref, cwa, cwb, b1_ref, dwa, dwb, b2_ref,
                o_ref, pf1_ref, y1_ref, pf2_ref):
    _conv1_patches(x_ref, pf1_ref)
    _tap_dots_epilogue(pf1_ref, cwa, cwb, b1_ref, y1_ref)
    _conv2_patches(y1_ref, pf2_ref)
    _tap_dots_epilogue(pf2_ref, dwa, dwb, b2_ref, o_ref)


def _conv_stages(x, w1mats, b1, w2mats, b2, ib):
    n_img = x.shape[0]
    bt1 = jnp.tile(b1, 8).reshape(1, 128).astype(jnp.float32)
    bt2 = jnp.tile(b2, 4).reshape(1, 128).astype(jnp.float32)
    return pl.pallas_call(
        _convs_body,
        out_shape=jax.ShapeDtypeStruct((n_img * 784, 128), jnp.bfloat16),
        grid=(n_img // ib,),
        in_specs=[pl.BlockSpec((ib, 3, 224, 224), lambda i: (i, 0, 0, 0))] + [
            pl.BlockSpec((216, 256), lambda i: (0, 0))] * 2 + [
            pl.BlockSpec((1, 128), lambda i: (0, 0))] + [
            pl.BlockSpec((640, 256), lambda i: (0, 0))] * 2 + [
            pl.BlockSpec((1, 128), lambda i: (0, 0)),
        ],
        out_specs=pl.BlockSpec((ib * 784, 128), lambda i: (i, 0)),
        scratch_shapes=[
            pltpu.VMEM((ib * 1568, 216), jnp.float32),
            pltpu.VMEM((ib * 1568, 128), jnp.bfloat16),
            pltpu.VMEM((ib * 784, 640), jnp.bfloat16),
        ],
        compiler_params=pltpu.CompilerParams(
            dimension_semantics=("parallel",)),
    )(x, *w1mats, bt1, *w2mats, bt2)


def _mlp_body(x_ref, w1_ref, b1_ref, w2_ref, b2_ref, o_ref, acc_ref):
    k = pl.program_id(0)

    @pl.when(k == 0)
    def _():
        acc_ref[...] = jnp.zeros_like(acc_ref)

    w1t = w1_ref[...].reshape(-1, w1_ref.shape[-1])
    acc_ref[...] += jnp.dot(x_ref[...].astype(jnp.float32), w1t,
                            preferred_element_type=jnp.float32)

    @pl.when(k == pl.num_programs(0) - 1)
    def _():
        h = jnp.maximum(acc_ref[...] + b1_ref[...], 0.0)
        o_ref[...] = (jnp.dot(h, w2_ref[...],
                              preferred_element_type=jnp.float32)
                      + b2_ref[...])


def _mlp_head(x, w1v, b1, w2, b2):
    N = x.shape[0]
    Hd = w1v.shape[-1]
    O = w2.shape[1]
    tk = 56 * 4 * 32
    return pl.pallas_call(
        _mlp_body,
        out_shape=jax.ShapeDtypeStruct((N, O), jnp.float32),
        grid=(14,),
        in_specs=[
            pl.BlockSpec((N, tk), lambda k: (0, k)),
            pl.BlockSpec((56, 1, 4, 32, Hd), lambda k: (0, k, 0, 0, 0)),
            pl.BlockSpec((1, Hd), lambda k: (0, 0)),
            pl.BlockSpec((Hd, O), lambda k: (0, 0)),
            pl.BlockSpec((1, O), lambda k: (0, 0)),
        ],
        out_specs=pl.BlockSpec((N, O), lambda k: (0, 0)),
        scratch_shapes=[pltpu.VMEM((N, Hd), jnp.float32)],
        compiler_params=pltpu.CompilerParams(
            dimension_semantics=("arbitrary",),
            vmem_limit_bytes=64 * 1024 * 1024,
        ),
    )(x, w1v, b1.reshape(1, Hd), w2, b2.reshape(1, O))


def kernel(x_nchw, conv1_w, conv1_b, conv2_w, conv2_b, fc1_w, fc1_b,
           fc2_w, fc2_b):
    N = x_nchw.shape[0]
    w1mats = _tap_weights(conv1_w, S=8, Cin=3, Cout=16, ci_major=True)
    w2mats = _tap_weights(conv2_w, S=4, Cin=16, Cout=32, ci_major=False)

    y2 = _conv_stages(x_nchw, w1mats, conv1_b, w2mats, conv2_b, ib=4)

    flat = y2.reshape(N, 56 * 56 * 32)
    w1v = fc1_w.reshape(56, 14, 4, 32, 128)
    return _mlp_head(flat, w1v, fc1_b, fc2_w, fc2_b)


# speedup vs baseline: 1.3018x; 1.3018x over previous
import numpy as np
import jax
import jax.numpy as jnp
from jax.experimental import pallas as pl
from jax.experimental.pallas import tpu as pltpu


def _tap_weights(w_mat, S, Cin, Cout, ci_major):
    W = 2 * S + 2
    Kp = 4 * W * Cin
    P = np.zeros((4, Kp, S * 9 * Cin), np.float32)
    for t, (dh, dw) in enumerate([(0, 0), (0, 1), (1, 0), (1, 1)]):
        for s in range(S):
            for kh in range(3):
                for kw in range(3):
                    for ci in range(Cin):
                        r = dh + kh
                        c = 2 * s + dw + kw
                        if ci_major:
                            dst = (r * Cin + ci) * W + c
                        else:
                            dst = (r * W + c) * Cin + ci
                        src = (kh * 3 + kw) * Cin + ci
                        P[t, dst, s * 9 * Cin + src] = 1.0
    wrep = (jnp.eye(S, dtype=w_mat.dtype)[:, None, :, None]
            * w_mat[None, :, None, :]).reshape(S * 9 * Cin, S * Cout)
    wall = jnp.einsum('tkm,mn->tkn', jnp.asarray(P), wrep).astype(jnp.bfloat16)
    return [jnp.concatenate([wall[0], wall[1]], axis=1),
            jnp.concatenate([wall[2], wall[3]], axis=1)]


def _tap_dots_epilogue(pf_ref, wa_ref, wb_ref, b_ref, o_ref):
    p = pf_ref[...]
    za = jnp.dot(p, wa_ref[...], preferred_element_type=jnp.float32)
    zb = jnp.dot(p, wb_ref[...], preferred_element_type=jnp.float32)
    z = jnp.maximum(za, zb)
    z = jnp.maximum(z[:, :128], z[:, 128:])
    o_ref[...] = jnp.maximum(z + b_ref[...], 0.0).astype(o_ref.dtype)


_PHASE1 = {(ph, r): ((2 * ph + r - 1) % 4, (2 * ph + r - 1 - (2 * ph + r - 1) % 4) // 4)
           for ph in range(2) for r in range(4)}
_PHASE2 = {r: ((r - 1) % 2, (r - 1 - (r - 1) % 2) // 2) for r in range(4)}


def _conv1_patches(x_ref, pf_ref):
    IB = x_ref.shape[0]
    pf_ref[...] = jnp.zeros_like(pf_ref)
    for img in range(IB):
        planes = {}
        for ci in range(3):
            pb = x_ref[img, ci].astype(jnp.bfloat16)
            pbm = jnp.concatenate(
                [pb, jnp.zeros((224, 32), jnp.bfloat16)], axis=1)
            pbm = pbm.reshape(56, 1024)
            for pp in range(4):
                planes[(ci, pp)] = pbm[:, pp * 256:pp * 256 + 224]
        for g in range(14):
            cs = 16 * g - 1
            src_lo, src_hi = max(cs, 0), min(cs + 18, 224)
            dst_lo = src_lo - cs
            for ph in range(2):
                for r in range(4):
                    pp, dlt = _PHASE1[(ph, r)]
                    if dlt == 0:
                        qs, qe, ds, de = 0, 56, 0, 56
                    elif dlt < 0:
                        qs, qe, ds, de = 0, 55, 1, 56
                    else:
                        qs, qe, ds, de = 1, 56, 0, 55
                    rb = img * 1568 + g * 112 + ph * 56
                    for ci in range(3):
                        lane = (r * 3 + ci) * 18 + dst_lo
                        pf_ref[rb + ds:rb + de, lane:lane + src_hi - src_lo] = \
                            planes[(ci, pp)][qs:qe, src_lo:src_hi]


def _conv2_patches(y_ref, pf_ref):
    IB = y_ref.shape[0] // 1568
    pf_ref[...] = jnp.zeros_like(pf_ref)
    for img in range(IB):
        for g2 in range(14):
            for r in range(4):
                ph, dlt = _PHASE2[r]
                if dlt == 0:
                    qs, qe, ds, de = 0, 56, 0, 56
                elif dlt < 0:
                    qs, qe, ds, de = 0, 55, 1, 56
                else:
                    qs, qe, ds, de = 1, 56, 0, 55
                rb = img * 784 + g2 * 56
                lane0 = r * 160
                pieces = []
                if g2 > 0:
                    pieces.append((g2 - 1, 112, 16, lane0))
                pieces.append((g2, 0, 128, lane0 + 16))
                if g2 < 13:
                    pieces.append((g2 + 1, 0, 16, lane0 + 144))
                for (gs, ls, lw, dl) in pieces:
                    sb = img * 1568 + gs * 112 + ph * 56
                    pf_ref[rb + ds:rb + de, dl:dl + lw] = \
                        y_ref[sb + qs:sb + qe, ls:ls + lw]


def _convs_body(x_ref, cwa, cwb, b1_ref, dwa, dwb, b2_ref,
                o_ref, pf1_ref, y1_ref, pf2_ref):
    _conv1_patches(x_ref, pf1_ref)
    _tap_dots_epilogue(pf1_ref, cwa, cwb, b1_ref, y1_ref)
    _conv2_patches(y1_ref, pf2_ref)
    _tap_dots_epilogue(pf2_ref, dwa, dwb, b2_ref, o_ref)


def _conv_stages(x, w1mats, b1, w2mats, b2, ib):
    n_img = x.shape[0]
    bt1 = jnp.tile(b1, 8).reshape(1, 128).astype(jnp.float32)
    bt2 = jnp.tile(b2, 4).reshape(1, 128).astype(jnp.float32)
    return pl.pallas_call(
        _convs_body,
        out_shape=jax.ShapeDtypeStruct((n_img * 784, 128), jnp.bfloat16),
        grid=(n_img // ib,),
        in_specs=[pl.BlockSpec((ib, 3, 224, 224), lambda i: (i, 0, 0, 0))] + [
            pl.BlockSpec((216, 256), lambda i: (0, 0))] * 2 + [
            pl.BlockSpec((1, 128), lambda i: (0, 0))] + [
            pl.BlockSpec((640, 256), lambda i: (0, 0))] * 2 + [
            pl.BlockSpec((1, 128), lambda i: (0, 0)),
        ],
        out_specs=pl.BlockSpec((ib * 784, 128), lambda i: (i, 0)),
        scratch_shapes=[
            pltpu.VMEM((ib * 1568, 216), jnp.bfloat16),
            pltpu.VMEM((ib * 1568, 128), jnp.bfloat16),
            pltpu.VMEM((ib * 784, 640), jnp.bfloat16),
        ],
        compiler_params=pltpu.CompilerParams(
            dimension_semantics=("parallel",)),
    )(x, *w1mats, bt1, *w2mats, bt2)


def _mlp_body(x_ref, w1_ref, b1_ref, w2_ref, b2_ref, o_ref, acc_ref):
    k = pl.program_id(0)

    @pl.when(k == 0)
    def _():
        acc_ref[...] = jnp.zeros_like(acc_ref)

    w1t = w1_ref[...].reshape(-1, w1_ref.shape[-1])
    acc_ref[...] += jnp.dot(x_ref[...].astype(jnp.float32), w1t,
                            preferred_element_type=jnp.float32)

    @pl.when(k == pl.num_programs(0) - 1)
    def _():
        h = jnp.maximum(acc_ref[...] + b1_ref[...], 0.0)
        o_ref[...] = (jnp.dot(h, w2_ref[...],
                              preferred_element_type=jnp.float32)
                      + b2_ref[...])


def _mlp_head(x, w1v, b1, w2, b2):
    N = x.shape[0]
    Hd = w1v.shape[-1]
    O = w2.shape[1]
    tk = 56 * 4 * 32
    return pl.pallas_call(
        _mlp_body,
        out_shape=jax.ShapeDtypeStruct((N, O), jnp.float32),
        grid=(14,),
        in_specs=[
            pl.BlockSpec((N, tk), lambda k: (0, k)),
            pl.BlockSpec((56, 1, 4, 32, Hd), lambda k: (0, k, 0, 0, 0)),
            pl.BlockSpec((1, Hd), lambda k: (0, 0)),
            pl.BlockSpec((Hd, O), lambda k: (0, 0)),
            pl.BlockSpec((1, O), lambda k: (0, 0)),
        ],
        out_specs=pl.BlockSpec((N, O), lambda k: (0, 0)),
        scratch_shapes=[pltpu.VMEM((N, Hd), jnp.float32)],
        compiler_params=pltpu.CompilerParams(
            dimension_semantics=("arbitrary",),
            vmem_limit_bytes=64 * 1024 * 1024,
        ),
    )(x, w1v, b1.reshape(1, Hd), w2, b2.reshape(1, O))


def kernel(x_nchw, conv1_w, conv1_b, conv2_w, conv2_b, fc1_w, fc1_b,
           fc2_w, fc2_b):
    N = x_nchw.shape[0]
    w1mats = _tap_weights(conv1_w, S=8, Cin=3, Cout=16, ci_major=True)
    w2mats = _tap_weights(conv2_w, S=4, Cin=16, Cout=32, ci_major=False)

    y2 = _conv_stages(x_nchw, w1mats, conv1_b, w2mats, conv2_b, ib=4)

    flat = y2.reshape(N, 56 * 56 * 32)
    w1v = fc1_w.reshape(56, 14, 4, 32, 128)
    return _mlp_head(flat, w1v, fc1_b, fc2_w, fc2_b)


# speedup vs baseline: 1.3075x; 1.0044x over previous
import numpy as np
import jax
import jax.numpy as jnp
from jax.experimental import pallas as pl
from jax.experimental.pallas import tpu as pltpu


def _tap_weights(w_mat, S, Cin, Cout, ci_major):
    W = 2 * S + 2
    Kp = 4 * W * Cin
    P = np.zeros((4, Kp, S * 9 * Cin), np.float32)
    for t, (dh, dw) in enumerate([(0, 0), (0, 1), (1, 0), (1, 1)]):
        for s in range(S):
            for kh in range(3):
                for kw in range(3):
                    for ci in range(Cin):
                        r = dh + kh
                        c = 2 * s + dw + kw
                        if ci_major:
                            dst = (r * Cin + ci) * W + c
                        else:
                            dst = (r * W + c) * Cin + ci
                        src = (kh * 3 + kw) * Cin + ci
                        P[t, dst, s * 9 * Cin + src] = 1.0
    wrep = (jnp.eye(S, dtype=w_mat.dtype)[:, None, :, None]
            * w_mat[None, :, None, :]).reshape(S * 9 * Cin, S * Cout)
    wall = jnp.einsum('tkm,mn->tkn', jnp.asarray(P), wrep).astype(jnp.bfloat16)
    return [jnp.concatenate([wall[0], wall[1]], axis=1),
            jnp.concatenate([wall[2], wall[3]], axis=1)]


def _tap_dots_epilogue(pf_ref, wa_ref, wb_ref, b_ref, o_ref):
    p = pf_ref[...]
    za = jnp.dot(p, wa_ref[...], preferred_element_type=jnp.float32)
    zb = jnp.dot(p, wb_ref[...], preferred_element_type=jnp.float32)
    z = jnp.maximum(za, zb)
    z = jnp.maximum(z[:, :128], z[:, 128:])
    o_ref[...] = jnp.maximum(z + b_ref[...], 0.0).astype(o_ref.dtype)


_PHASE1 = {(ph, r): ((2 * ph + r - 1) % 4, (2 * ph + r - 1 - (2 * ph + r - 1) % 4) // 4)
           for ph in range(2) for r in range(4)}
_PHASE2 = {r: ((r - 1) % 2, (r - 1 - (r - 1) % 2) // 2) for r in range(4)}


def _conv1_patches(x_ref, pf_ref):
    IB = x_ref.shape[0]
    pf_ref[...] = jnp.zeros_like(pf_ref)
    for img in range(IB):
        planes = {}
        for ci in range(3):
            pb = x_ref[img, ci].astype(jnp.bfloat16)
            pbm = jnp.concatenate(
                [pb, jnp.zeros((224, 32), jnp.bfloat16)], axis=1)
            pbm = pbm.reshape(56, 1024)
            for pp in range(4):
                planes[(ci, pp)] = pbm[:, pp * 256:pp * 256 + 224]
        for g in range(14):
            cs = 16 * g - 1
            src_lo, src_hi = max(cs, 0), min(cs + 18, 224)
            dst_lo = src_lo - cs
            for ph in range(2):
                for r in range(4):
                    pp, dlt = _PHASE1[(ph, r)]
                    if dlt == 0:
                        qs, qe, ds, de = 0, 56, 0, 56
                    elif dlt < 0:
                        qs, qe, ds, de = 0, 55, 1, 56
                    else:
                        qs, qe, ds, de = 1, 56, 0, 55
                    rb = img * 1568 + g * 112 + ph * 56
                    for ci in range(3):
                        lane = (r * 3 + ci) * 18 + dst_lo
                        pf_ref[rb + ds:rb + de, lane:lane + src_hi - src_lo] = \
                            planes[(ci, pp)][qs:qe, src_lo:src_hi]


def _conv2_patches(y_ref, pf_ref):
    IB = y_ref.shape[0] // 1568
    pf_ref[...] = jnp.zeros_like(pf_ref)
    for img in range(IB):
        for g2 in range(14):
            for r in range(4):
                ph, dlt = _PHASE2[r]
                if dlt == 0:
                    qs, qe, ds, de = 0, 56, 0, 56
                elif dlt < 0:
                    qs, qe, ds, de = 0, 55, 1, 56
                else:
                    qs, qe, ds, de = 1, 56, 0, 55
                rb = img * 784 + g2 * 56
                lane0 = r * 160
                pieces = []
                if g2 > 0:
                    pieces.append((g2 - 1, 112, 16, lane0))
                pieces.append((g2, 0, 128, lane0 + 16))
                if g2 < 13:
                    pieces.append((g2 + 1, 0, 16, lane0 + 144))
                for (gs, ls, lw, dl) in pieces:
                    sb = img * 1568 + gs * 112 + ph * 56
                    pf_ref[rb + ds:rb + de, dl:dl + lw] = \
                        y_ref[sb + qs:sb + qe, ls:ls + lw]


def _convs_body(x_ref, cwa, cwb, b1_ref, dwa, dwb, b2_ref,
                o_ref, pf1_ref, y1_ref, pf2_ref):
    _conv1_patches(x_ref, pf1_ref)
    _tap_dots_epilogue(pf1_ref, cwa, cwb, b1_ref, y1_ref)
    _conv2_patches(y1_ref, pf2_ref)
    _tap_dots_epilogue(pf2_ref, dwa, dwb, b2_ref, o_ref)


def _conv_stages(x, w1mats, b1, w2mats, b2, ib):
    n_img = x.shape[0]
    bt1 = jnp.tile(b1, 8).reshape(1, 128).astype(jnp.float32)
    bt2 = jnp.tile(b2, 4).reshape(1, 128).astype(jnp.float32)
    return pl.pallas_call(
        _convs_body,
        out_shape=jax.ShapeDtypeStruct((n_img * 784, 128), jnp.bfloat16),
        grid=(n_img // ib,),
        in_specs=[pl.BlockSpec((ib, 3, 224, 224), lambda i: (i, 0, 0, 0))] + [
            pl.BlockSpec((216, 256), lambda i: (0, 0))] * 2 + [
            pl.BlockSpec((1, 128), lambda i: (0, 0))] + [
            pl.BlockSpec((640, 256), lambda i: (0, 0))] * 2 + [
            pl.BlockSpec((1, 128), lambda i: (0, 0)),
        ],
        out_specs=pl.BlockSpec((ib * 784, 128), lambda i: (i, 0)),
        scratch_shapes=[
            pltpu.VMEM((ib * 1568, 216), jnp.bfloat16),
            pltpu.VMEM((ib * 1568, 128), jnp.bfloat16),
            pltpu.VMEM((ib * 784, 640), jnp.bfloat16),
        ],
        compiler_params=pltpu.CompilerParams(
            dimension_semantics=("parallel",)),
    )(x, *w1mats, bt1, *w2mats, bt2)


def _mlp_body(x_ref, w1_ref, b1_ref, w2_ref, b2_ref, o_ref, acc_ref):
    k = pl.program_id(0)

    @pl.when(k == 0)
    def _():
        acc_ref[...] = jnp.zeros_like(acc_ref)

    w1t = w1_ref[...].reshape(-1, w1_ref.shape[-1])
    acc_ref[...] += jnp.dot(x_ref[...].astype(jnp.float32), w1t,
                            preferred_element_type=jnp.float32)

    @pl.when(k == pl.num_programs(0) - 1)
    def _():
        h = jnp.maximum(acc_ref[...] + b1_ref[...], 0.0)
        o_ref[...] = (jnp.dot(h, w2_ref[...],
                              preferred_element_type=jnp.float32)
                      + b2_ref[...])


def _mlp_head(x, w1v, b1, w2, b2):
    N = x.shape[0]
    Hd = w1v.shape[-1]
    O = w2.shape[1]
    tk = 56 * 4 * 32
    return pl.pallas_call(
        _mlp_body,
        out_shape=jax.ShapeDtypeStruct((N, O), jnp.float32),
        grid=(14,),
        in_specs=[
            pl.BlockSpec((N, tk), lambda k: (0, k)),
            pl.BlockSpec((56, 1, 4, 32, Hd), lambda k: (0, k, 0, 0, 0)),
            pl.BlockSpec((1, Hd), lambda k: (0, 0)),
            pl.BlockSpec((Hd, O), lambda k: (0, 0)),
            pl.BlockSpec((1, O), lambda k: (0, 0)),
        ],
        out_specs=pl.BlockSpec((N, O), lambda k: (0, 0)),
        scratch_shapes=[pltpu.VMEM((N, Hd), jnp.float32)],
        compiler_params=pltpu.CompilerParams(
            dimension_semantics=("arbitrary",),
            vmem_limit_bytes=64 * 1024 * 1024,
        ),
    )(x, w1v, b1.reshape(1, Hd), w2, b2.reshape(1, O))


def kernel(x_nchw, conv1_w, conv1_b, conv2_w, conv2_b, fc1_w, fc1_b,
           fc2_w, fc2_b):
    N = x_nchw.shape[0]
    w1mats = _tap_weights(conv1_w, S=8, Cin=3, Cout=16, ci_major=True)
    w2mats = _tap_weights(conv2_w, S=4, Cin=16, Cout=32, ci_major=False)

    y2 = _conv_stages(x_nchw, w1mats, conv1_b, w2mats, conv2_b, ib=8)

    flat = y2.reshape(N, 56 * 56 * 32)
    w1v = fc1_w.reshape(56, 14, 4, 32, 128)
    return _mlp_head(flat, w1v, fc1_b, fc2_w, fc2_b)
